# Initial kernel scaffold; baseline (speedup 1.0000x reference)
import jax
import jax.numpy as jnp
from jax.experimental import pallas as pl


def kernel(input_, weight, scales_stacked, qweight_stacked, qzeros_stacked, indices):
    raise NotImplementedError("write your pallas kernel here")


# speedup vs baseline: 2.0546x; 2.0546x over previous
import functools

import jax
import jax.numpy as jnp
from jax import lax
from jax.experimental import pallas as pl
from jax.experimental.pallas import tpu as pltpu

IN_F = 4096
OUT_F = 4096
N_EXP = 8
PACK = 8
N_TOK = 32
BLK_O = 512


def _body(x_ref, xp_ref, idx_ref, w_ref, q_ref, z_ref, s_ref, o_ref):
    e = pl.program_id(1)

    q = q_ref[0]
    parts = [q & 15]
    for n in range(1, PACK - 1):
        parts.append((q >> (4 * n)) & 15)
    parts.append(q >> (4 * (PACK - 1)))
    u = jnp.concatenate(parts, axis=1).astype(jnp.bfloat16)

    mask = idx_ref[...] == e
    xm = jnp.where(mask, xp_ref[...], jnp.bfloat16(0))
    dot = lax.dot_general(
        xm, u, (((1,), (1,)), ((), ())), preferred_element_type=jnp.float32
    )
    rs = jnp.sum(xm.astype(jnp.float32), axis=1, keepdims=True)
    delta = (dot - rs * z_ref[0]) * s_ref[0]

    @pl.when(e == 0)
    def _():
        wb = w_ref[...].astype(jnp.bfloat16)
        base = lax.dot_general(
            x_ref[...], wb, (((1,), (1,)), ((), ())),
            preferred_element_type=jnp.float32,
        )
        o_ref[...] = base + delta

    @pl.when(e != 0)
    def _():
        o_ref[...] += delta


@jax.jit
def _run(x, xp, idx, weight, qweight, z, s):
    grid = (OUT_F // BLK_O, N_EXP)
    return pl.pallas_call(
        _body,
        grid=grid,
        in_specs=[
            pl.BlockSpec((N_TOK, IN_F), lambda o, e: (0, 0)),
            pl.BlockSpec((N_TOK, IN_F), lambda o, e: (0, 0)),
            pl.BlockSpec((N_TOK, 1), lambda o, e: (0, 0)),
            pl.BlockSpec((BLK_O, IN_F), lambda o, e: (o, 0)),
            pl.BlockSpec((1, BLK_O, IN_F // PACK), lambda o, e: (e, o, 0)),
            pl.BlockSpec((1, 1, BLK_O), lambda o, e: (e, 0, o)),
            pl.BlockSpec((1, 1, BLK_O), lambda o, e: (e, 0, o)),
        ],
        out_specs=pl.BlockSpec((N_TOK, BLK_O), lambda o, e: (0, o)),
        out_shape=jax.ShapeDtypeStruct((N_TOK, OUT_F), jnp.float32),
        compiler_params=pltpu.CompilerParams(
            dimension_semantics=("parallel", "arbitrary"),
        ),
    )(x, xp, idx, weight, qweight, z, s)


def kernel(input_, weight, scales_stacked, qweight_stacked, qzeros_stacked, indices):
    x = input_.astype(jnp.bfloat16)
    xp = (
        input_.reshape(N_TOK, IN_F // PACK, PACK)
        .transpose(0, 2, 1)
        .reshape(N_TOK, IN_F)
        .astype(jnp.bfloat16)
    )
    idx = indices.reshape(N_TOK, 1)
    qz = qzeros_stacked.reshape(N_EXP, OUT_F // PACK)
    shifts = jnp.arange(PACK, dtype=jnp.int32) * 4
    z = ((qz[:, :, None] >> shifts) & 15).astype(jnp.float32).reshape(
        N_EXP, 1, OUT_F
    )
    s = scales_stacked.reshape(N_EXP, 1, OUT_F)
    return _run(x, xp, idx, weight, qweight_stacked, z, s)
